# baseline (device time: 53761 ns/iter reference)
import jax
import jax.numpy as jnp
from jax import lax
from jax.experimental import pallas as pl
from jax.experimental.pallas import tpu as pltpu

N_DEV = 4
NP = 2


def kernel(A, B, stage="full"):
    m, _ = A.shape
    _, n = B.shape
    chunk = m // N_DEV
    half = n // 2
    piece = chunk // NP

    def body(
        a_ref,
        b_ref,
        out_ref,
        a_vm, b_vm,
        b_bf,
        e1b, e2b, e3b, e4b, mlloc, mrloc, ownL, ownR,
        chainL, chainR, dirL, dirR, sumL, sumR, msendL, msendR,
        in_sems,
        e1ss, e1rs, e2ss, e2rs, e3ss, e3rs, e4ss, e4rs,
        m1ss, m1rs, m2ss, m2rs,
        ag_ssem_r, ag_rsem_r, ag_ssem_l, ag_rsem_l,
    ):
        my = lax.axis_index("i")
        left = (my - 1) % N_DEV
        right = (my + 1) % N_DEV

        b_copy = pltpu.make_async_copy(b_ref, b_vm, in_sems.at[0])
        a_copy = pltpu.make_async_copy(a_ref, a_vm, in_sems.at[1])
        b_copy.start()
        a_copy.start()

        barrier_sem = pltpu.get_barrier_semaphore()
        for nbr in (left, right):
            pl.semaphore_signal(
                barrier_sem,
                inc=1,
                device_id=(nbr,),
                device_id_type=pl.DeviceIdType.MESH,
            )
        pl.semaphore_wait(barrier_sem, 2)
        b_copy.wait()
        a_copy.wait()

        def rdma(src, dst, ssem, rsem, target):
            return pltpu.make_async_remote_copy(
                src_ref=src, dst_ref=dst, send_sem=ssem, recv_sem=rsem,
                device_id=(target,), device_id_type=pl.DeviceIdType.MESH,
            )

        E1 = [rdma(e1b.at[p], chainL.at[p], e1ss.at[p], e1rs.at[p], right)
              for p in range(NP)]
        E2 = [rdma(e2b.at[p], chainR.at[p], e2ss.at[p], e2rs.at[p], left)
              for p in range(NP)]
        E3 = [rdma(e3b.at[p], dirL.at[p], e3ss.at[p], e3rs.at[p], left)
              for p in range(NP)]
        E4 = [rdma(e4b.at[p], dirR.at[p], e4ss.at[p], e4rs.at[p], right)
              for p in range(NP)]
        M1 = [rdma(msendL.at[p], sumL.at[p], m1ss.at[p], m1rs.at[p], right)
              for p in range(NP)]
        M2 = [rdma(msendR.at[p], sumR.at[p], m2ss.at[p], m2rs.at[p], left)
              for p in range(NP)]

        def out_sl(c, p, col0):
            return out_ref.at[
                pl.ds(c * chunk + p * piece, piece), pl.ds(col0, half)
            ]

        ag_send_r = [[rdma(out_sl((my - s) % N_DEV, p, 0),
                           out_sl((my - s) % N_DEV, p, 0),
                           ag_ssem_r.at[s, p], ag_rsem_r.at[s, p], right)
                      for p in range(NP)] for s in range(3)]
        ag_recv_r = [[rdma(out_sl((my - 1 - s) % N_DEV, p, 0),
                           out_sl((my - 1 - s) % N_DEV, p, 0),
                           ag_ssem_r.at[s, p], ag_rsem_r.at[s, p], right)
                      for p in range(NP)] for s in range(3)]
        ag_send_l = [[rdma(out_sl((my + s) % N_DEV, p, half),
                           out_sl((my + s) % N_DEV, p, half),
                           ag_ssem_l.at[s, p], ag_rsem_l.at[s, p], left)
                      for p in range(NP)] for s in range(3)]
        ag_recv_l = [[rdma(out_sl((my + 1 + s) % N_DEV, p, half),
                           out_sl((my + 1 + s) % N_DEV, p, half),
                           ag_ssem_l.at[s, p], ag_rsem_l.at[s, p], left)
                      for p in range(NP)] for s in range(3)]

        def dot_into(dst, c, p, col0):
            ap = a_vm[pl.ds(c * chunk + p * piece, piece), :].astype(
                jnp.bfloat16
            )
            dst[p, :, :] = jnp.dot(
                ap, b_bf[:, pl.ds(col0, half)],
                preferred_element_type=jnp.float32,
            ).astype(jnp.bfloat16)

        comm = stage not in ("mm", "esend_only", "noop")
        esend = stage in ("esend", "esend_only") or comm

        if stage == "noop":
            out_ref[pl.ds(0, piece), pl.ds(0, half)] = b_bf[
                pl.ds(0, piece), pl.ds(0, half)
            ]
            return

        b_bf[:, :half] = b_vm[:, :half].astype(jnp.bfloat16)
        for p in range(NP):
            dot_into(e1b, (my + 2) % N_DEV, p, 0)
            if esend:
                E1[p].start()
        b_bf[:, half:] = b_vm[:, half:].astype(jnp.bfloat16)
        for p in range(NP):
            dot_into(e2b, (my + 2) % N_DEV, p, half)
            if esend:
                E2[p].start()
        for p in range(NP):
            dot_into(e4b, (my + 1) % N_DEV, p, half)
            if esend:
                E4[p].start()
        for p in range(NP):
            dot_into(e3b, (my - 1) % N_DEV, p, 0)
            if esend:
                E3[p].start()

        for p in range(NP):
            dot_into(mlloc, (my + 1) % N_DEV, p, 0)
            if comm:
                E1[p].wait_recv()
                msendL[p, :, :] = mlloc[p, :, :] + chainL[p, :, :]
                M1[p].start()
        for p in range(NP):
            dot_into(mrloc, (my - 1) % N_DEV, p, half)
            if comm:
                E2[p].wait_recv()
                msendR[p, :, :] = mrloc[p, :, :] + chainR[p, :, :]
                M2[p].start()

        for p in range(NP):
            dot_into(ownL, my, p, 0)
            if comm:
                M1[p].wait_recv()
                E3[p].wait_recv()
                zL = (
                    ownL[p, :, :].astype(jnp.float32)
                    + sumL[p, :, :].astype(jnp.float32)
                    + dirL[p, :, :].astype(jnp.float32)
                )
                out_ref[pl.ds(my * chunk + p * piece, piece), pl.ds(0, half)] = (
                    zL / (1.0 + jnp.exp(-zL))
                ).astype(jnp.bfloat16)
                if stage == "full":
                    ag_send_r[0][p].start()
        for p in range(NP):
            dot_into(ownR, my, p, half)
            if comm:
                M2[p].wait_recv()
                E4[p].wait_recv()
                zR = (
                    ownR[p, :, :].astype(jnp.float32)
                    + sumR[p, :, :].astype(jnp.float32)
                    + dirR[p, :, :].astype(jnp.float32)
                )
                out_ref[
                    pl.ds(my * chunk + p * piece, piece), pl.ds(half, half)
                ] = (zR / (1.0 + jnp.exp(-zR))).astype(jnp.bfloat16)
                if stage == "full":
                    ag_send_l[0][p].start()

        if stage == "esend_only":
            for flow in (E1, E2, E3, E4):
                for op in flow:
                    op.wait_send()
            return
        if stage == "mm":
            return

        if stage == "full":
            for s in range(3):
                for p in range(NP):
                    ag_recv_r[s][p].wait_recv()
                    if s < 2:
                        ag_send_r[s + 1][p].start()
                    ag_recv_l[s][p].wait_recv()
                    if s < 2:
                        ag_send_l[s + 1][p].start()

        for flow in (E1, E2, E3, E4, M1, M2):
            for op in flow:
                op.wait_send()
        if stage == "full":
            for grid in (ag_send_r, ag_send_l):
                for ops in grid:
                    for op in ops:
                        op.wait_send()

    return pl.pallas_call(
        body,
        out_shape=jax.ShapeDtypeStruct((m, n), jnp.bfloat16),
        in_specs=[
            pl.BlockSpec(memory_space=pl.ANY),
            pl.BlockSpec(memory_space=pl.ANY),
        ],
        out_specs=pl.BlockSpec(memory_space=pltpu.VMEM),
        scratch_shapes=[
            pltpu.VMEM(A.shape, jnp.float32),
            pltpu.VMEM(B.shape, jnp.float32),
            pltpu.VMEM(B.shape, jnp.bfloat16),
            pltpu.VMEM((NP, piece, half), jnp.bfloat16),
            pltpu.VMEM((NP, piece, half), jnp.bfloat16),
            pltpu.VMEM((NP, piece, half), jnp.bfloat16),
            pltpu.VMEM((NP, piece, half), jnp.bfloat16),
            pltpu.VMEM((NP, piece, half), jnp.bfloat16),
            pltpu.VMEM((NP, piece, half), jnp.bfloat16),
            pltpu.VMEM((NP, piece, half), jnp.bfloat16),
            pltpu.VMEM((NP, piece, half), jnp.bfloat16),
            pltpu.VMEM((NP, piece, half), jnp.bfloat16),
            pltpu.VMEM((NP, piece, half), jnp.bfloat16),
            pltpu.VMEM((NP, piece, half), jnp.bfloat16),
            pltpu.VMEM((NP, piece, half), jnp.bfloat16),
            pltpu.VMEM((NP, piece, half), jnp.bfloat16),
            pltpu.VMEM((NP, piece, half), jnp.bfloat16),
            pltpu.VMEM((NP, piece, half), jnp.bfloat16),
            pltpu.VMEM((NP, piece, half), jnp.bfloat16),
            pltpu.SemaphoreType.DMA((2,)),
            pltpu.SemaphoreType.DMA((NP,)),
            pltpu.SemaphoreType.DMA((NP,)),
            pltpu.SemaphoreType.DMA((NP,)),
            pltpu.SemaphoreType.DMA((NP,)),
            pltpu.SemaphoreType.DMA((NP,)),
            pltpu.SemaphoreType.DMA((NP,)),
            pltpu.SemaphoreType.DMA((NP,)),
            pltpu.SemaphoreType.DMA((NP,)),
            pltpu.SemaphoreType.DMA((NP,)),
            pltpu.SemaphoreType.DMA((NP,)),
            pltpu.SemaphoreType.DMA((NP,)),
            pltpu.SemaphoreType.DMA((NP,)),
            pltpu.SemaphoreType.DMA((3, NP)),
            pltpu.SemaphoreType.DMA((3, NP)),
            pltpu.SemaphoreType.DMA((3, NP)),
            pltpu.SemaphoreType.DMA((3, NP)),
        ],
        compiler_params=pltpu.CompilerParams(collective_id=0),
    )(A, B)


# device time: 49980 ns/iter; 1.0757x vs baseline; 1.0757x over previous
import jax
import jax.numpy as jnp
from jax import lax
from jax.experimental import pallas as pl
from jax.experimental.pallas import tpu as pltpu

N_DEV = 4
NP = 2


def kernel(A, B, stage="full"):
    m, _ = A.shape
    _, n = B.shape
    chunk = m // N_DEV
    half = n // 2
    piece = chunk // NP

    def body(
        a_ref,
        b_ref,
        out_ref,
        b_bf,
        pcL, pcR,
        rs_send_r, rs_recv_r, rs_send_l, rs_recv_l,
        rs_ssem_r, rs_rsem_r, rs_ssem_l, rs_rsem_l,
        ag_ssem_r, ag_rsem_r, ag_ssem_l, ag_rsem_l,
    ):
        my = lax.axis_index("i")
        left = (my - 1) % N_DEV
        right = (my + 1) % N_DEV

        barrier_sem = pltpu.get_barrier_semaphore()
        for nbr in (left, right):
            pl.semaphore_signal(
                barrier_sem,
                inc=1,
                device_id=(nbr,),
                device_id_type=pl.DeviceIdType.MESH,
            )
        pl.semaphore_wait(barrier_sem, 2)

        b_bf[:, :half] = b_ref[:, :half].astype(jnp.bfloat16)

        def rdma(src, dst, ssem, rsem, target):
            return pltpu.make_async_remote_copy(
                src_ref=src, dst_ref=dst, send_sem=ssem, recv_sem=rsem,
                device_id=(target,), device_id_type=pl.DeviceIdType.MESH,
            )

        rs_r = [[rdma(rs_send_r.at[s, p], rs_recv_r.at[s, p],
                      rs_ssem_r.at[s, p], rs_rsem_r.at[s, p], right)
                 for p in range(NP)] for s in range(3)]
        rs_l = [[rdma(rs_send_l.at[s, p], rs_recv_l.at[s, p],
                      rs_ssem_l.at[s, p], rs_rsem_l.at[s, p], left)
                 for p in range(NP)] for s in range(3)]

        def out_sl(c, p, col0):
            return out_ref.at[
                pl.ds(c * chunk + p * piece, piece), pl.ds(col0, half)
            ]

        ag_send_r = [[rdma(out_sl((my + 1 - s) % N_DEV, p, 0),
                           out_sl((my + 1 - s) % N_DEV, p, 0),
                           ag_ssem_r.at[s, p], ag_rsem_r.at[s, p], right)
                      for p in range(NP)] for s in range(3)]
        ag_recv_r = [[rdma(out_sl((my - s) % N_DEV, p, 0),
                           out_sl((my - s) % N_DEV, p, 0),
                           ag_ssem_r.at[s, p], ag_rsem_r.at[s, p], right)
                      for p in range(NP)] for s in range(3)]
        ag_send_l = [[rdma(out_sl((my - 1 + s) % N_DEV, p, half),
                           out_sl((my - 1 + s) % N_DEV, p, half),
                           ag_ssem_l.at[s, p], ag_rsem_l.at[s, p], left)
                      for p in range(NP)] for s in range(3)]
        ag_recv_l = [[rdma(out_sl((my + s) % N_DEV, p, half),
                           out_sl((my + s) % N_DEV, p, half),
                           ag_ssem_l.at[s, p], ag_rsem_l.at[s, p], left)
                      for p in range(NP)] for s in range(3)]

        def dotL(c):
            ac = a_ref[pl.ds(c * chunk, chunk), :].astype(jnp.bfloat16)
            pcL[pl.ds(c * chunk, chunk), :] = jnp.dot(
                ac, b_bf[:, :half], preferred_element_type=jnp.float32
            ).astype(jnp.bfloat16)

        def dotR(c):
            ac = a_ref[pl.ds(c * chunk, chunk), :].astype(jnp.bfloat16)
            pcR[pl.ds(c * chunk, chunk), :] = jnp.dot(
                ac, b_bf[:, half:], preferred_element_type=jnp.float32
            ).astype(jnp.bfloat16)

        for p in range(NP):
            ap = a_ref[
                pl.ds(my * chunk + p * piece, piece), :
            ].astype(jnp.bfloat16)
            rs_send_r[0, p, :, :] = jnp.dot(
                ap, b_bf[:, :half], preferred_element_type=jnp.float32
            ).astype(jnp.bfloat16)
            if stage != "mm":
                rs_r[0][p].start()
            if p == 0:
                b_bf[:, half:] = b_ref[:, half:].astype(jnp.bfloat16)
            rs_send_l[0, p, :, :] = jnp.dot(
                ap, b_bf[:, half:], preferred_element_type=jnp.float32
            ).astype(jnp.bfloat16)
            if stage != "mm":
                rs_l[0][p].start()

        if stage == "mm":
            for c in ((my - 1) % N_DEV, (my + 1) % N_DEV, (my + 2) % N_DEV):
                dotL(c)
                dotR(c)
            out_ref[pl.ds(0, piece), pl.ds(0, half)] = rs_send_r[0, 0, :, :]
            return

        def rs_step(s):
            cr = (my - s - 1) % N_DEV
            cl = (my + s + 1) % N_DEV
            for p in range(NP):
                rs_r[s][p].wait_recv()
                rs_send_r[s + 1, p, :, :] = (
                    pcL[pl.ds(cr * chunk + p * piece, piece), :]
                    + rs_recv_r[s, p, :, :]
                )
                rs_r[s + 1][p].start()
                rs_l[s][p].wait_recv()
                rs_send_l[s + 1, p, :, :] = (
                    pcR[pl.ds(cl * chunk + p * piece, piece), :]
                    + rs_recv_l[s, p, :, :]
                )
                rs_l[s + 1][p].start()

        dotL((my - 1) % N_DEV)
        dotR((my + 1) % N_DEV)
        rs_step(0)
        dotL((my + 2) % N_DEV)
        dotR((my + 2) % N_DEV)
        rs_step(1)
        dotL((my + 1) % N_DEV)
        dotR((my - 1) % N_DEV)

        own_r = (my + 1) % N_DEV
        own_l = (my - 1) % N_DEV
        for p in range(NP):
            rs_r[2][p].wait_recv()
            zr = pcL[
                pl.ds(own_r * chunk + p * piece, piece), :
            ].astype(jnp.float32) + rs_recv_r[2, p, :, :].astype(jnp.float32)
            out_ref[
                pl.ds(own_r * chunk + p * piece, piece), pl.ds(0, half)
            ] = (zr / (1.0 + jnp.exp(-zr))).astype(jnp.bfloat16)
            if stage == "full":
                ag_send_r[0][p].start()
            rs_l[2][p].wait_recv()
            zl = pcR[
                pl.ds(own_l * chunk + p * piece, piece), :
            ].astype(jnp.float32) + rs_recv_l[2, p, :, :].astype(jnp.float32)
            out_ref[
                pl.ds(own_l * chunk + p * piece, piece), pl.ds(half, half)
            ] = (zl / (1.0 + jnp.exp(-zl))).astype(jnp.bfloat16)
            if stage == "full":
                ag_send_l[0][p].start()

        if stage == "full":
            for s in range(3):
                for p in range(NP):
                    ag_recv_r[s][p].wait_recv()
                    if s < 2:
                        ag_send_r[s + 1][p].start()
                    ag_recv_l[s][p].wait_recv()
                    if s < 2:
                        ag_send_l[s + 1][p].start()

        grids = (rs_r, rs_l) if stage != "full" else (
            rs_r, rs_l, ag_send_r, ag_send_l
        )
        for grid in grids:
            for ops in grid:
                for op in ops:
                    op.wait_send()

    nhop = N_DEV - 1
    return pl.pallas_call(
        body,
        out_shape=jax.ShapeDtypeStruct((m, n), jnp.bfloat16),
        in_specs=[
            pl.BlockSpec(memory_space=pltpu.VMEM),
            pl.BlockSpec(memory_space=pltpu.VMEM),
        ],
        out_specs=pl.BlockSpec(memory_space=pltpu.VMEM),
        scratch_shapes=[
            pltpu.VMEM(B.shape, jnp.bfloat16),
            pltpu.VMEM((m, half), jnp.bfloat16),
            pltpu.VMEM((m, half), jnp.bfloat16),
            pltpu.VMEM((nhop, NP, piece, half), jnp.bfloat16),
            pltpu.VMEM((nhop, NP, piece, half), jnp.bfloat16),
            pltpu.VMEM((nhop, NP, piece, half), jnp.bfloat16),
            pltpu.VMEM((nhop, NP, piece, half), jnp.bfloat16),
            pltpu.SemaphoreType.DMA((nhop, NP)),
            pltpu.SemaphoreType.DMA((nhop, NP)),
            pltpu.SemaphoreType.DMA((nhop, NP)),
            pltpu.SemaphoreType.DMA((nhop, NP)),
            pltpu.SemaphoreType.DMA((nhop, NP)),
            pltpu.SemaphoreType.DMA((nhop, NP)),
            pltpu.SemaphoreType.DMA((nhop, NP)),
            pltpu.SemaphoreType.DMA((nhop, NP)),
        ],
        compiler_params=pltpu.CompilerParams(collective_id=0),
    )(A, B)
